# baseline (speedup 1.0000x reference)
import functools

import jax
import jax.numpy as jnp
import numpy as np
from jax import lax
from jax.experimental import pallas as pl
from jax.experimental.pallas import tpu as pltpu
from jax.experimental.pallas import tpu_sc as plsc

B, S, CLASS_NUM, E_DIMS, LEVELS = 4096, 50, 100000, 256, 128
N_TOK = B * S
N_CLS = (S + 7) // 8

_T_OVER_PI = float((2.0 ** 30) / np.pi)
_TWO64 = float(2.0 ** 64)

_D1 = 6.283185307179586
_D3 = -41.34170224039975
_D5 = 81.60524927607504
_D7 = -76.70585975306136
_D9 = 42.05869394489765

_ROWS_BLK = 16
_TOK_BLK = _ROWS_BLK * S


def _make_consts():
    lane = np.arange(256)
    k = lane >> 1
    parity = lane & 1
    pk = np.ldexp(np.float32(1.0), k).astype(np.float32)
    offh = 0.25 * parity.astype(np.float32)
    thr = np.where(k == 127, -1.0, _T_OVER_PI).astype(np.float32)
    return jnp.asarray(np.stack([pk, offh, thr]))


def _enc_body(x_ref, c_ref, g_ref, o_ref):
    j = pl.program_id(1)
    v = x_ref[...]
    minv = jnp.min(v)
    compute = jnp.logical_or(j == 0, minv * _TWO64 <= _T_OVER_PI)

    @pl.when(compute)
    def _():
        pk = c_ref[0:1, :]
        offh = c_ref[1:2, :]
        thr = c_ref[2:3, :]
        w = v * pk
        h = w * 0.5
        f = h - jnp.floor(h)
        f2 = f + offh
        t = (f2 + f2) + 0.5
        q = jnp.minimum(jnp.floor(t), 2.0)
        g = f2 - q * 0.5
        g2 = g * g
        p = (((_D9 * g2 + _D7) * g2 + _D5) * g2 + _D3) * g2 + _D1
        res = p * g
        res = jnp.where(q == 1.0, -res, res)
        res = jnp.where(w > thr, jnp.float32(jnp.nan), res)
        o_ref[...] = res

    @pl.when(jnp.logical_not(compute))
    def _():
        o_ref[...] = jnp.full((_TOK_BLK, 128), jnp.nan, jnp.float32)

    for jr in range(_ROWS_BLK):
        for c in range(N_CLS):
            o_ref[pl.ds(S * jr + 8 * c, 1), :] = g_ref[c, pl.ds(jr, 1), :]


_enc_call = pl.pallas_call(
    _enc_body,
    grid=(N_TOK // _TOK_BLK, 2),
    in_specs=[
        pl.BlockSpec((_TOK_BLK, 1), lambda i, j: (i, jnp.int32(0))),
        pl.BlockSpec((3, 128), lambda i, j: (jnp.int32(0), j)),
        pl.BlockSpec((N_CLS, _ROWS_BLK, 128),
                     lambda i, j: (jnp.int32(0), i, j)),
    ],
    out_specs=pl.BlockSpec((_TOK_BLK, 128), lambda i, j: (i, j)),
    out_shape=jax.ShapeDtypeStruct((N_TOK, 256), jnp.float32),
    compiler_params=pltpu.CompilerParams(
        dimension_semantics=("arbitrary", "arbitrary"),
    ),
)


def _sc_body(x_ref, e_ref, g_ref, xv, idx_v, rows_v, sem_g, sem_s):
    nc = 2
    wid = lax.axis_index("s") * nc + lax.axis_index("c")
    rows_per_w = B // 32
    base = wid * rows_per_w

    pltpu.sync_copy(x_ref.at[pl.ds(base * S, rows_per_w * S)], xv)

    lanes = lax.broadcasted_iota(jnp.int32, (16,), 0)

    def chunk(c, buf):
        for g in range(rows_per_w // 16):
            rows16 = lanes + 16 * g
            vals = plsc.load_gather(xv, [rows16 * S + 8 * c])
            idx_v[buf][pl.ds(16 * g, 16)] = vals.astype(jnp.int32)

    stores = [None, None]
    for c in range(N_CLS):
        buf = c % 2
        if stores[buf] is not None:
            stores[buf].wait()
        chunk(c, buf)
        pltpu.async_copy(e_ref.at[idx_v[buf]], rows_v[buf], sem_g).wait()
        stores[buf] = pltpu.async_copy(
            rows_v[buf], g_ref.at[jnp.int32(c), pl.ds(base, rows_per_w)],
            sem_s)
    for st in stores:
        if st is not None:
            st.wait()


@functools.cache
def _get_sc_call():
    return pl.kernel(
        _sc_body,
        out_type=jax.ShapeDtypeStruct((N_CLS, B, E_DIMS), jnp.float32),
        mesh=plsc.VectorSubcoreMesh(core_axis_name="c", subcore_axis_name="s"),
        compiler_params=pltpu.CompilerParams(needs_layout_passes=False),
        scratch_types=dict(
            xv=pltpu.VMEM((B // 32 * S,), jnp.float32),
            idx_v=[pltpu.VMEM((B // 32,), jnp.int32) for _ in range(2)],
            rows_v=[pltpu.VMEM((B // 32, E_DIMS), jnp.float32)
                    for _ in range(2)],
            sem_g=pltpu.SemaphoreType.DMA,
            sem_s=pltpu.SemaphoreType.DMA,
        ),
    )


def kernel(x, E_class):
    x_flat = x.reshape(N_TOK)
    g = _get_sc_call()(x_flat, E_class)
    out = _enc_call(x_flat.reshape(N_TOK, 1), _make_consts(), g)
    return out.reshape(B, S, 256)


# speedup vs baseline: 1.3542x; 1.3542x over previous
import functools

import jax
import jax.numpy as jnp
import numpy as np
from jax import lax
from jax.experimental import pallas as pl
from jax.experimental.pallas import tpu as pltpu
from jax.experimental.pallas import tpu_sc as plsc

B, S, CLASS_NUM, E_DIMS, LEVELS = 4096, 50, 100000, 256, 128
N_TOK = B * S
N_CLS = (S + 7) // 8

_T_OVER_PI = float((2.0 ** 30) / np.pi)
_TWO64 = float(2.0 ** 64)

_D1 = 6.283185307179586
_D3 = -41.34170224039975
_D5 = 81.60524927607504
_D7 = -76.70585975306136
_D9 = 42.05869394489765

_ROWS_BLK = 16
_TOK_BLK = _ROWS_BLK * S


def _make_consts():
    lane = np.arange(256)
    k = lane >> 1
    parity = lane & 1
    pk = np.ldexp(np.float32(1.0), k).astype(np.float32)
    offh = 0.25 * parity.astype(np.float32)
    thr = np.where(k == 127, -1.0, _T_OVER_PI).astype(np.float32)
    return jnp.asarray(np.stack([pk, offh, thr]))


def _sincos(v, pk, offh, thr):
    w = v * pk
    h = w * 0.5
    f = h - jnp.floor(h)
    f2 = f + offh
    t = (f2 + f2) + 0.5
    q = jnp.minimum(jnp.floor(t), 2.0)
    g = f2 - q * 0.5
    g2 = g * g
    p = (((_D9 * g2 + _D7) * g2 + _D5) * g2 + _D3) * g2 + _D1
    res = p * g
    res = jnp.where(q == 1.0, -res, res)
    return jnp.where(w > thr, jnp.float32(jnp.nan), res)


def _enc_body(x_ref, x2_ref, c_ref, g_ref, o_ref):
    v = x_ref[...]
    o_ref[:, 0:128] = _sincos(v, c_ref[0:1, 0:128], c_ref[1:2, 0:128],
                              jnp.float32(_T_OVER_PI))
    minv = jnp.min(x2_ref[...])
    skip = minv * _TWO64 > _T_OVER_PI

    @pl.when(skip)
    def _():
        o_ref[:, 128:256] = jnp.full((_TOK_BLK, 128), jnp.nan, jnp.float32)

    @pl.when(jnp.logical_not(skip))
    def _():
        o_ref[:, 128:256] = _sincos(v, c_ref[0:1, 128:256],
                                    c_ref[1:2, 128:256], c_ref[2:3, 128:256])

    for jr in range(_ROWS_BLK):
        for c in range(N_CLS):
            o_ref[pl.ds(S * jr + 8 * c, 1), :] = g_ref[c, pl.ds(jr, 1), :]


_enc_call = pl.pallas_call(
    _enc_body,
    grid=(N_TOK // _TOK_BLK,),
    in_specs=[
        pl.BlockSpec((_TOK_BLK, 1), lambda i: (i, jnp.int32(0))),
        pl.BlockSpec((_ROWS_BLK, S), lambda i: (i, jnp.int32(0))),
        pl.BlockSpec((3, 256), lambda i: (jnp.int32(0), jnp.int32(0))),
        pl.BlockSpec((N_CLS, _ROWS_BLK, 256),
                     lambda i: (jnp.int32(0), i, jnp.int32(0))),
    ],
    out_specs=pl.BlockSpec((_TOK_BLK, 256), lambda i: (i, jnp.int32(0))),
    out_shape=jax.ShapeDtypeStruct((N_TOK, 256), jnp.float32),
    compiler_params=pltpu.CompilerParams(
        dimension_semantics=("arbitrary",),
    ),
)


def _sc_body(x_ref, e_ref, g_ref, xv, idx_v, rows_v, sem_g, sem_s):
    nc = 2
    wid = lax.axis_index("s") * nc + lax.axis_index("c")
    rows_per_w = B // 32
    base = wid * rows_per_w

    pltpu.sync_copy(x_ref.at[pl.ds(base * S, rows_per_w * S)], xv)

    lanes = lax.broadcasted_iota(jnp.int32, (16,), 0)

    def chunk(c, buf):
        for g in range(rows_per_w // 16):
            rows16 = lanes + 16 * g
            vals = plsc.load_gather(xv, [rows16 * S + 8 * c])
            idx_v[buf][pl.ds(16 * g, 16)] = vals.astype(jnp.int32)

    stores = [None, None]
    for c in range(N_CLS):
        buf = c % 2
        if stores[buf] is not None:
            stores[buf].wait()
        chunk(c, buf)
        pltpu.async_copy(e_ref.at[idx_v[buf]], rows_v[buf], sem_g).wait()
        stores[buf] = pltpu.async_copy(
            rows_v[buf], g_ref.at[jnp.int32(c), pl.ds(base, rows_per_w)],
            sem_s)
    for st in stores:
        if st is not None:
            st.wait()


@functools.cache
def _get_sc_call():
    return pl.kernel(
        _sc_body,
        out_type=jax.ShapeDtypeStruct((N_CLS, B, E_DIMS), jnp.float32),
        mesh=plsc.VectorSubcoreMesh(core_axis_name="c", subcore_axis_name="s"),
        compiler_params=pltpu.CompilerParams(needs_layout_passes=False),
        scratch_types=dict(
            xv=pltpu.VMEM((B // 32 * S,), jnp.float32),
            idx_v=[pltpu.VMEM((B // 32,), jnp.int32) for _ in range(2)],
            rows_v=[pltpu.VMEM((B // 32, E_DIMS), jnp.float32)
                    for _ in range(2)],
            sem_g=pltpu.SemaphoreType.DMA,
            sem_s=pltpu.SemaphoreType.DMA,
        ),
    )


def kernel(x, E_class):
    x_flat = x.reshape(N_TOK)
    g = _get_sc_call()(x_flat, E_class)
    out = _enc_call(x_flat.reshape(N_TOK, 1), x, _make_consts(), g)
    return out.reshape(B, S, 256)


# speedup vs baseline: 2.2446x; 1.6575x over previous
import functools

import jax
import jax.numpy as jnp
import numpy as np
from jax import lax
from jax.experimental import pallas as pl
from jax.experimental.pallas import tpu as pltpu
from jax.experimental.pallas import tpu_sc as plsc

B, S, CLASS_NUM, E_DIMS, LEVELS = 4096, 50, 100000, 256, 128
N_TOK = B * S
N_CLS = (S + 7) // 8

_T_OVER_PI = float((2.0 ** 30) / np.pi)
_TWO64 = float(2.0 ** 64)

_D1 = 6.283185307179586
_D3 = -41.34170224039975
_D5 = 81.60524927607504
_D7 = -76.70585975306136
_D9 = 42.05869394489765

_ROWS_BLK = 16
_TOK_BLK = _ROWS_BLK * S


def _make_consts():
    lane = np.arange(256)
    k = lane >> 1
    parity = lane & 1
    pk = np.ldexp(np.float32(1.0), k).astype(np.float32)
    offh = 0.25 * parity.astype(np.float32)
    thr = np.where(k == 127, -1.0, _T_OVER_PI).astype(np.float32)
    return jnp.asarray(np.stack([pk, offh, thr]))


def _sincos(v, pk, offh, thr):
    w = v * pk
    h = w * 0.5
    f = h - jnp.floor(h)
    f2 = f + offh
    t = (f2 + f2) + 0.5
    q = jnp.minimum(jnp.floor(t), 2.0)
    g = f2 - q * 0.5
    g2 = g * g
    p = (((_D9 * g2 + _D7) * g2 + _D5) * g2 + _D3) * g2 + _D1
    res = p * g
    res = jnp.where(q == 1.0, -res, res)
    return jnp.where(w > thr, jnp.float32(jnp.nan), res)


def _enc_body(x2_ref, c_ref, g_ref, o_ref):
    v2 = x2_ref[...]
    v = jnp.reshape(v2, (_ROWS_BLK, S, 1))
    pk = jnp.reshape(c_ref[0:1, :], (1, 1, 256))
    offh = jnp.reshape(c_ref[1:2, :], (1, 1, 256))
    thr = jnp.reshape(c_ref[2:3, :], (1, 1, 256))
    o_ref[:, :, 0:128] = _sincos(v, pk[:, :, 0:128], offh[:, :, 0:128],
                                 jnp.float32(_T_OVER_PI))
    minv = jnp.min(v2)
    skip = minv * _TWO64 > _T_OVER_PI

    @pl.when(skip)
    def _():
        o_ref[:, :, 128:256] = jnp.full((_ROWS_BLK, S, 128), jnp.nan,
                                        jnp.float32)

    @pl.when(jnp.logical_not(skip))
    def _():
        o_ref[:, :, 128:256] = _sincos(v, pk[:, :, 128:256],
                                       offh[:, :, 128:256],
                                       thr[:, :, 128:256])

    for jr in range(_ROWS_BLK):
        for c in range(N_CLS):
            o_ref[pl.ds(jr, 1), pl.ds(8 * c, 1), :] = jnp.reshape(
                g_ref[c, pl.ds(jr, 1), :], (1, 1, 256))


_enc_call = pl.pallas_call(
    _enc_body,
    grid=(B // _ROWS_BLK,),
    in_specs=[
        pl.BlockSpec((_ROWS_BLK, S), lambda i: (i, jnp.int32(0))),
        pl.BlockSpec((3, 256), lambda i: (jnp.int32(0), jnp.int32(0))),
        pl.BlockSpec((N_CLS, _ROWS_BLK, 256),
                     lambda i: (jnp.int32(0), i, jnp.int32(0))),
    ],
    out_specs=pl.BlockSpec((_ROWS_BLK, S, 256),
                           lambda i: (i, jnp.int32(0), jnp.int32(0))),
    out_shape=jax.ShapeDtypeStruct((B, S, 256), jnp.float32),
    compiler_params=pltpu.CompilerParams(
        dimension_semantics=("arbitrary",),
    ),
)


def _sc_body(x_ref, e_ref, g_ref, xv, idx_v, rows_v, sem_g, sem_s):
    nc = 2
    wid = lax.axis_index("s") * nc + lax.axis_index("c")
    rows_per_w = B // 32
    base = wid * rows_per_w

    pltpu.sync_copy(x_ref.at[pl.ds(base * S, rows_per_w * S)], xv)

    lanes = lax.broadcasted_iota(jnp.int32, (16,), 0)

    def chunk(c, buf):
        for g in range(rows_per_w // 16):
            rows16 = lanes + 16 * g
            vals = plsc.load_gather(xv, [rows16 * S + 8 * c])
            idx_v[buf][pl.ds(16 * g, 16)] = vals.astype(jnp.int32)

    stores = [None, None]
    for c in range(N_CLS):
        buf = c % 2
        if stores[buf] is not None:
            stores[buf].wait()
        chunk(c, buf)
        pltpu.async_copy(e_ref.at[idx_v[buf]], rows_v[buf], sem_g).wait()
        stores[buf] = pltpu.async_copy(
            rows_v[buf], g_ref.at[jnp.int32(c), pl.ds(base, rows_per_w)],
            sem_s)
    for st in stores:
        if st is not None:
            st.wait()


@functools.cache
def _get_sc_call():
    return pl.kernel(
        _sc_body,
        out_type=jax.ShapeDtypeStruct((N_CLS, B, E_DIMS), jnp.float32),
        mesh=plsc.VectorSubcoreMesh(core_axis_name="c", subcore_axis_name="s"),
        compiler_params=pltpu.CompilerParams(needs_layout_passes=False),
        scratch_types=dict(
            xv=pltpu.VMEM((B // 32 * S,), jnp.float32),
            idx_v=[pltpu.VMEM((B // 32,), jnp.int32) for _ in range(2)],
            rows_v=[pltpu.VMEM((B // 32, E_DIMS), jnp.float32)
                    for _ in range(2)],
            sem_g=pltpu.SemaphoreType.DMA,
            sem_s=pltpu.SemaphoreType.DMA,
        ),
    )


def kernel(x, E_class):
    g = _get_sc_call()(x.reshape(N_TOK), E_class)
    return _enc_call(x, _make_consts(), g)
